# baseline (speedup 1.0000x reference)
import jax
import jax.numpy as jnp
from jax.experimental import pallas as pl
from jax.experimental.pallas import tpu as pltpu

_NUM_NETS = 8
_NUM_PROXIES = 100000
_EMBED_DIM = 16
_B = 26
_BLK = 25000
_NR = _NUM_PROXIES // _BLK


def _body(scond_ref, perm_ref, sadj_ref, t_ref, z_ref, x_ref):
    z_ref[...] = t_ref[...]
    r = pl.program_id(0)
    i = pl.program_id(1)
    a = sadj_ref[i]
    b = perm_ref[i]
    lo = r * _BLK

    @pl.when((a >= lo) & (a < lo + _BLK))
    def _():
        x_ref[pl.ds(b, 1), :] = t_ref[pl.ds(a - lo, 1), :]


def kernel(tables, cond, adjs):
    perm = jnp.argsort(cond).astype(jnp.int32)
    scond = cond[perm]
    sadj = adjs[perm]

    grid_spec = pltpu.PrefetchScalarGridSpec(
        num_scalar_prefetch=3,
        grid=(_NR, _B),
        in_specs=[
            pl.BlockSpec(
                (None, _BLK, _EMBED_DIM),
                lambda r, i, sc, pm, sa: (sc[i], r, 0),
            ),
        ],
        out_specs=[
            pl.BlockSpec(
                (None, _BLK, _EMBED_DIM),
                lambda r, i, sc, pm, sa: (pm[i], r, 0),
            ),
            pl.BlockSpec((_B, _EMBED_DIM), lambda r, i, sc, pm, sa: (0, 0)),
        ],
    )

    z, x = pl.pallas_call(
        _body,
        grid_spec=grid_spec,
        out_shape=[
            jax.ShapeDtypeStruct((_B, _NUM_PROXIES, _EMBED_DIM), jnp.float32),
            jax.ShapeDtypeStruct((_B, _EMBED_DIM), jnp.float32),
        ],
        compiler_params=pltpu.CompilerParams(
            dimension_semantics=("arbitrary", "arbitrary"),
        ),
    )(scond, perm, sadj, tables)
    return (x, z)
